# Initial kernel scaffold; baseline (speedup 1.0000x reference)
import functools

import jax
import jax.numpy as jnp
from jax import lax
from jax.experimental import pallas as pl
from jax.experimental.pallas import tpu as pltpu
from jax.experimental.pallas import tpu_sc as plsc

_NUM_CORES = 2
_NUM_SUBCORES = 16
_NW = _NUM_CORES * _NUM_SUBCORES


@functools.lru_cache(maxsize=None)
def _make_gather(B: int, D: int, chunk: int):
    assert B % _NW == 0
    bpw = B // _NW
    assert bpw % chunk == 0
    nchunk = bpw // chunk
    mesh = plsc.VectorSubcoreMesh(core_axis_name="c", subcore_axis_name="s")

    @functools.partial(
        pl.kernel,
        mesh=mesh,
        out_type=jax.ShapeDtypeStruct((B, D), jnp.float32),
        scratch_types=[
            pltpu.VMEM((chunk,), jnp.int32),
            pltpu.VMEM((chunk, D), jnp.float32),
            pltpu.SemaphoreType.DMA,
        ],
    )
    def gather(idx_hbm, table_hbm, out_hbm, idx_v, rows_v, sem):
        wid = lax.axis_index("s") * _NUM_CORES + lax.axis_index("c")
        base = wid * bpw

        def body(i, carry):
            off = base + i * chunk
            pltpu.sync_copy(idx_hbm.at[pl.ds(off, chunk)], idx_v)
            pltpu.async_copy(table_hbm.at[idx_v], rows_v, sem).wait()
            pltpu.sync_copy(rows_v, out_hbm.at[pl.ds(off, chunk)])
            return carry

        lax.fori_loop(0, nchunk, body, 0)

    return gather


def kernel(x, table):
    bq, hist = x.shape
    b = bq * hist
    idx = x.reshape(b).astype(jnp.int32)
    out = _make_gather(b, table.shape[1], 512)(idx, table)
    return out.reshape(bq, hist, table.shape[1])


# speedup vs baseline: 3.9534x; 3.9534x over previous
import functools

import jax
import jax.numpy as jnp
from jax import lax
from jax.experimental import pallas as pl
from jax.experimental.pallas import tpu as pltpu
from jax.experimental.pallas import tpu_sc as plsc

_NUM_CORES = 2
_NUM_SUBCORES = 16
_NW = _NUM_CORES * _NUM_SUBCORES


@functools.lru_cache(maxsize=None)
def _make_gather(B: int, D: int, chunk: int):
    assert B % _NW == 0
    bpw = B // _NW
    assert bpw % chunk == 0
    nchunk = bpw // chunk
    mesh = plsc.VectorSubcoreMesh(core_axis_name="c", subcore_axis_name="s")

    @functools.partial(
        pl.kernel,
        mesh=mesh,
        out_type=jax.ShapeDtypeStruct((B, D), jnp.float32),
        scratch_types=[
            pltpu.VMEM((chunk,), jnp.int32),
            pltpu.VMEM((chunk, D), jnp.float32),
            pltpu.SemaphoreType.DMA,
        ],
        compiler_params=pltpu.CompilerParams(use_tc_tiling_on_sc=False),
    )
    def gather(idx_hbm, table_hbm, out_hbm, idx_v, rows_v, sem):
        wid = lax.axis_index("s") * _NUM_CORES + lax.axis_index("c")
        base = wid * bpw

        def body(i, carry):
            off = base + i * chunk
            pltpu.sync_copy(idx_hbm.at[pl.ds(off, chunk)], idx_v)
            pltpu.async_copy(table_hbm.at[idx_v], rows_v, sem).wait()
            pltpu.sync_copy(rows_v, out_hbm.at[pl.ds(off, chunk)])
            return carry

        lax.fori_loop(0, nchunk, body, 0)

    return gather


def kernel(x, table):
    bq, hist = x.shape
    b = bq * hist
    idx = x.reshape(b).astype(jnp.int32)
    out = _make_gather(b, table.shape[1], 512)(idx, table)
    return out.reshape(bq, hist, table.shape[1])


# speedup vs baseline: 4.2754x; 1.0815x over previous
import functools

import jax
import jax.numpy as jnp
from jax import lax
from jax.experimental import pallas as pl
from jax.experimental.pallas import tpu as pltpu
from jax.experimental.pallas import tpu_sc as plsc

_NUM_CORES = 2
_NUM_SUBCORES = 16
_NW = _NUM_CORES * _NUM_SUBCORES
_NBUF = 4


@functools.lru_cache(maxsize=None)
def _make_gather(B: int, D: int, chunk: int):
    assert B % _NW == 0
    bpw = B // _NW
    assert bpw % chunk == 0
    nchunk = bpw // chunk
    assert nchunk % _NBUF == 0 and nchunk >= 2 * _NBUF
    mesh = plsc.VectorSubcoreMesh(core_axis_name="c", subcore_axis_name="s")

    @functools.partial(
        pl.kernel,
        mesh=mesh,
        out_type=jax.ShapeDtypeStruct((B, D), jnp.float32),
        scratch_types=[
            pltpu.VMEM((bpw,), jnp.int32),
            [pltpu.VMEM((chunk, D), jnp.float32) for _ in range(_NBUF)],
            pltpu.SemaphoreType.DMA,
            pltpu.SemaphoreType.DMA,
        ],
        compiler_params=pltpu.CompilerParams(use_tc_tiling_on_sc=False),
    )
    def gather(idx_hbm, table_hbm, out_hbm, idx_v, bufs, gsem, wsem):
        wid = lax.axis_index("s") * _NUM_CORES + lax.axis_index("c")
        base = wid * bpw
        pltpu.sync_copy(idx_hbm.at[pl.ds(base, bpw)], idx_v)

        def g_start(c, buf):
            pltpu.async_copy(table_hbm.at[idx_v.at[pl.ds(c * chunk, chunk)]],
                             buf, gsem)

        def g_wait(buf):
            pltpu.make_async_copy(table_hbm.at[idx_v.at[pl.ds(0, chunk)]],
                                  buf, gsem).wait()

        def w_start(c, buf):
            pltpu.async_copy(buf, out_hbm.at[pl.ds(base + c * chunk, chunk)],
                             wsem)

        def w_wait(buf):
            pltpu.make_async_copy(buf, out_hbm.at[pl.ds(base, chunk)],
                                  wsem).wait()

        for b in range(_NBUF):
            g_start(b, bufs[b])

        def body(j, carry):
            for b in range(_NBUF):
                c = j * _NBUF + b
                g_wait(bufs[b])
                w_start(c, bufs[b])
                w_wait(bufs[b])
                g_start(c + _NBUF, bufs[b])
            return carry

        lax.fori_loop(0, nchunk // _NBUF - 1, body, 0)

        for b in range(_NBUF):
            c = nchunk - _NBUF + b
            g_wait(bufs[b])
            w_start(c, bufs[b])
        for b in range(_NBUF):
            w_wait(bufs[b])

    return gather


def kernel(x, table):
    bq, hist = x.shape
    b = bq * hist
    idx = x.reshape(b).astype(jnp.int32)
    out = _make_gather(b, table.shape[1], 256)(idx, table)
    return out.reshape(bq, hist, table.shape[1])


# speedup vs baseline: 4.6826x; 1.0952x over previous
import functools

import jax
import jax.numpy as jnp
from jax import lax
from jax.experimental import pallas as pl
from jax.experimental.pallas import tpu as pltpu
from jax.experimental.pallas import tpu_sc as plsc

_NUM_CORES = 2
_NUM_SUBCORES = 16
_NW = _NUM_CORES * _NUM_SUBCORES
_LANES = 16
_ROWS_PER_STEP = 8


@functools.lru_cache(maxsize=None)
def _make_gather(BQ: int, H: int, D: int):
    assert BQ % _NW == 0
    rows_pw = BQ // _NW
    bpw = rows_pw * H
    assert rows_pw % 2 == 0 and rows_pw >= 4
    assert H % _ROWS_PER_STEP == 0 and D % _LANES == 0
    mesh = plsc.VectorSubcoreMesh(core_axis_name="c", subcore_axis_name="s")

    @functools.partial(
        pl.kernel,
        mesh=mesh,
        out_type=jax.ShapeDtypeStruct((BQ, H, D), jnp.float32),
        scratch_types=[
            pltpu.VMEM((bpw,), jnp.int32),
            [pltpu.VMEM((H, 2 * D), jnp.float32) for _ in range(2)],
            [pltpu.VMEM((H, D), jnp.float32) for _ in range(2)],
            pltpu.SemaphoreType.DMA,
            pltpu.SemaphoreType.DMA,
        ],
        compiler_params=pltpu.CompilerParams(use_tc_tiling_on_sc=True),
    )
    def gather(idx_hbm, table_hbm, out_hbm, idx_v, bufa, bufb, gsem, wsem):
        wid = lax.axis_index("s") * _NUM_CORES + lax.axis_index("c")
        base = wid * bpw
        row0 = wid * rows_pw
        pltpu.sync_copy(idx_hbm.at[pl.ds(base, bpw)], idx_v)

        def g_start(c, a):
            pltpu.async_copy(table_hbm.at[idx_v.at[pl.ds(c * H, H)]],
                             a, gsem)

        def g_wait(a):
            pltpu.make_async_copy(table_hbm.at[idx_v.at[pl.ds(0, H)]],
                                  a, gsem).wait()

        def w_start(c, b):
            pltpu.async_copy(b, out_hbm.at[row0 + c], wsem)

        def w_wait(b):
            pltpu.make_async_copy(b, out_hbm.at[row0], wsem).wait()

        def vcopy(a, b):
            def rows(i, carry):
                r = i * _ROWS_PER_STEP
                for j in range(_ROWS_PER_STEP):
                    for k in range(D // _LANES):
                        b.at[r + j, pl.ds(k * _LANES, _LANES)][...] = (
                            a.at[r + j, pl.ds(k * _LANES, _LANES)][...])
                return carry
            lax.fori_loop(0, H // _ROWS_PER_STEP, rows, 0)

        g_start(0, bufa[0])
        g_start(1, bufa[1])
        for p in range(2):
            g_wait(bufa[p])
            vcopy(bufa[p], bufb[p])
            g_start(2 + p, bufa[p])
            w_start(p, bufb[p])

        def body(j, carry):
            for p in range(2):
                c = 2 * j + p
                g_wait(bufa[p])
                w_wait(bufb[p])
                vcopy(bufa[p], bufb[p])
                g_start(c + 2, bufa[p])
                w_start(c, bufb[p])
            return carry

        lax.fori_loop(1, rows_pw // 2 - 1, body, 0)

        for p in range(2):
            c = rows_pw - 2 + p
            g_wait(bufa[p])
            w_wait(bufb[p])
            vcopy(bufa[p], bufb[p])
            w_start(c, bufb[p])
        for p in range(2):
            w_wait(bufb[p])

    return gather


def kernel(x, table):
    bq, hist = x.shape
    d = table.shape[1]
    idx = x.reshape(bq * hist).astype(jnp.int32)
    table_pad = jnp.pad(table, ((0, 0), (0, d)))
    return _make_gather(bq, hist, d)(idx, table_pad)
